# Initial kernel scaffold; baseline (speedup 1.0000x reference)
import functools

import jax
import jax.numpy as jnp
from jax import lax
from jax.experimental import pallas as pl
from jax.experimental.pallas import tpu as pltpu
from jax.experimental.pallas import tpu_sc as plsc

N = 10000
D_IN, D_H, D_OUT = 128, 64, 16
NC, NS = 2, 16
NW = NC * NS
CH = 128
K = 8
N_PAD = 10112
ROWS_BLK = 1000

_mesh = functools.partial(
    plsc.VectorSubcoreMesh, core_axis_name="c", subcore_axis_name="s")


def _worker_prolog(zeros_hbm, acc):
    s = lax.axis_index("s")
    zchunk = N_PAD // NS
    pltpu.sync_copy(zeros_hbm.at[pl.ds(s * zchunk, zchunk)],
                    acc.at[pl.ds(s * zchunk, zchunk)])
    plsc.subcore_barrier()


def _worker_epilog(acc, out_hbm):
    c = lax.axis_index("c")
    s = lax.axis_index("s")
    zchunk = N_PAD // NS
    plsc.subcore_barrier()
    pltpu.sync_copy(acc.at[pl.ds(s * zchunk, zchunk)],
                    out_hbm.at[c, pl.ds(s * zchunk, zchunk)])


def _make_deg(rows_per_worker):
    @functools.partial(
        pl.kernel,
        out_type=jax.ShapeDtypeStruct((NC, N_PAD), jnp.float32),
        mesh=_mesh(),
        scratch_types=[
            pltpu.VMEM((K, CH), jnp.int32),
            pltpu.VMEM((CH,), jnp.float32),
            pltpu.VMEM_SHARED((N_PAD,), jnp.float32),
        ],
    )
    def deg_kernel(col2d, zeros1, degp, cidx, ones_v, acc):
        c = lax.axis_index("c")
        s = lax.axis_index("s")
        wid = s * NC + c
        for i in range(CH // 16):
            ones_v[pl.ds(i * 16, 16)] = jnp.ones((16,), jnp.float32)
        _worker_prolog(zeros1, acc)
        base = wid * rows_per_worker

        def step(t, carry):
            pltpu.sync_copy(col2d.at[pl.ds(base + t * K, K)], cidx)
            for j in range(K):
                pltpu.sync_copy(ones_v, acc.at[cidx.at[j]], add=True)
            return carry

        lax.fori_loop(0, rows_per_worker // K, step, 0)
        _worker_epilog(acc, degp)

    return deg_kernel


def _make_msg(width, rows_per_worker):
    @functools.partial(
        pl.kernel,
        out_type=jax.ShapeDtypeStruct((NC, N_PAD, width), jnp.float32),
        mesh=_mesh(),
        scratch_types=[
            pltpu.VMEM((K, CH), jnp.int32),
            pltpu.VMEM((K, CH), jnp.int32),
            pltpu.VMEM((2, CH, width), jnp.float32),
            pltpu.VMEM_SHARED((N_PAD, width), jnp.float32),
            pltpu.SemaphoreType.DMA,
            pltpu.SemaphoreType.DMA,
        ],
    )
    def msg_kernel(table, row2d, col2d, zerosw, raw,
                   ridx, cidx, gbuf, acc, sem0, sem1):
        c = lax.axis_index("c")
        s = lax.axis_index("s")
        wid = s * NC + c
        _worker_prolog(zerosw, acc)
        base = wid * rows_per_worker
        sems = [sem0, sem1]

        def step(t, carry):
            pltpu.sync_copy(row2d.at[pl.ds(base + t * K, K)], ridx)
            pltpu.sync_copy(col2d.at[pl.ds(base + t * K, K)], cidx)
            handles = [None, None]
            handles[0] = pltpu.async_copy(
                table.at[ridx.at[0]], gbuf.at[0], sems[0])
            for j in range(K):
                if j + 1 < K:
                    handles[(j + 1) % 2] = pltpu.async_copy(
                        table.at[ridx.at[j + 1]], gbuf.at[(j + 1) % 2],
                        sems[(j + 1) % 2])
                handles[j % 2].wait()
                pltpu.sync_copy(gbuf.at[j % 2], acc.at[cidx.at[j]], add=True)
            return carry

        lax.fori_loop(0, rows_per_worker // K, step, 0)
        _worker_epilog(acc, raw)

    return msg_kernel


def _tc1_body(x_ref, w1_ref, degp_ref, hs1_ref, dis_ref):
    deg = degp_ref[0] + degp_ref[1] + 1.0
    dis = lax.rsqrt(deg)
    h = jnp.dot(x_ref[...], w1_ref[...], preferred_element_type=jnp.float32)
    hs1_ref[...] = dis * h
    dis_ref[...] = dis


def _tc2_body(raw1_ref, hs1_ref, dis_ref, b1_ref, w2_ref, hs2_ref):
    dis = dis_ref[...]
    h = jnp.maximum(
        dis * (raw1_ref[0] + raw1_ref[1] + hs1_ref[...]) + b1_ref[...], 0.0)
    hs2_ref[...] = dis * jnp.dot(
        h, w2_ref[...], preferred_element_type=jnp.float32)


def _tc3_body(raw2_ref, hs2_ref, dis_ref, b2_ref, out_ref):
    z = (dis_ref[...] * (raw2_ref[0] + raw2_ref[1] + hs2_ref[...])
         + b2_ref[...])
    m = jnp.max(z, axis=1, keepdims=True)
    ssum = jnp.sum(jnp.exp(z - m), axis=1, keepdims=True)
    out_ref[...] = z - m - jnp.log(ssum)


def _tc1(x, W1, degp_col):
    grid = N // ROWS_BLK
    return pl.pallas_call(
        _tc1_body,
        grid=(grid,),
        in_specs=[
            pl.BlockSpec((ROWS_BLK, D_IN), lambda i: (i, 0)),
            pl.BlockSpec((D_IN, D_H), lambda i: (0, 0)),
            pl.BlockSpec((NC, ROWS_BLK, 1), lambda i: (0, i, 0)),
        ],
        out_specs=[
            pl.BlockSpec((ROWS_BLK, D_H), lambda i: (i, 0)),
            pl.BlockSpec((ROWS_BLK, 1), lambda i: (i, 0)),
        ],
        out_shape=[
            jax.ShapeDtypeStruct((N, D_H), jnp.float32),
            jax.ShapeDtypeStruct((N, 1), jnp.float32),
        ],
    )(x, W1, degp_col)


def _tc2(raw1, hs1, dis, b1, W2):
    grid = N // ROWS_BLK
    return pl.pallas_call(
        _tc2_body,
        grid=(grid,),
        in_specs=[
            pl.BlockSpec((NC, ROWS_BLK, D_H), lambda i: (0, i, 0)),
            pl.BlockSpec((ROWS_BLK, D_H), lambda i: (i, 0)),
            pl.BlockSpec((ROWS_BLK, 1), lambda i: (i, 0)),
            pl.BlockSpec((1, D_H), lambda i: (0, 0)),
            pl.BlockSpec((D_H, D_OUT), lambda i: (0, 0)),
        ],
        out_specs=pl.BlockSpec((ROWS_BLK, D_OUT), lambda i: (i, 0)),
        out_shape=jax.ShapeDtypeStruct((N, D_OUT), jnp.float32),
    )(raw1, hs1, dis, b1, W2)


def _tc3(raw2, hs2, dis, b2):
    grid = N // ROWS_BLK
    return pl.pallas_call(
        _tc3_body,
        grid=(grid,),
        in_specs=[
            pl.BlockSpec((NC, ROWS_BLK, D_OUT), lambda i: (0, i, 0)),
            pl.BlockSpec((ROWS_BLK, D_OUT), lambda i: (i, 0)),
            pl.BlockSpec((ROWS_BLK, 1), lambda i: (i, 0)),
            pl.BlockSpec((1, D_OUT), lambda i: (0, 0)),
        ],
        out_specs=pl.BlockSpec((ROWS_BLK, D_OUT), lambda i: (i, 0)),
        out_shape=jax.ShapeDtypeStruct((N, D_OUT), jnp.float32),
    )(raw2, hs2, dis, b2)


def kernel(x, edge_index, W1, b1, W2, b2):
    row = edge_index[0]
    col = edge_index[1]
    e = row.shape[0]
    quantum = NW * K * CH
    e_pad = ((e + quantum - 1) // quantum) * quantum
    pad = e_pad - e
    rowp = jnp.concatenate([row, jnp.zeros((pad,), row.dtype)])
    colp = jnp.concatenate([col, jnp.full((pad,), N, col.dtype)])
    row2d = rowp.reshape(-1, CH)
    col2d = colp.reshape(-1, CH)
    rows_per_worker = row2d.shape[0] // NW

    z1 = jnp.zeros((N_PAD,), jnp.float32)
    z64 = jnp.zeros((N_PAD, D_H), jnp.float32)
    z16 = jnp.zeros((N_PAD, D_OUT), jnp.float32)

    degp = _make_deg(rows_per_worker)(col2d, z1)
    degp_col = degp.reshape(NC, N_PAD, 1)[:, :N]

    hs1, dis = _tc1(x, W1, degp_col)
    raw1 = _make_msg(D_H, rows_per_worker)(hs1, row2d, col2d, z64)[:, :N]
    hs2 = _tc2(raw1, hs1, dis, b1.reshape(1, D_H), W2)
    raw2 = _make_msg(D_OUT, rows_per_worker)(hs2, row2d, col2d, z16)[:, :N]
    return _tc3(raw2, hs2, dis, b2.reshape(1, D_OUT))


# speedup vs baseline: 19.8753x; 19.8753x over previous
import functools

import jax
import jax.numpy as jnp
from jax import lax
from jax.experimental import pallas as pl
from jax.experimental.pallas import tpu as pltpu
from jax.experimental.pallas import tpu_sc as plsc

N = 10000
D_IN, D_H, D_OUT = 128, 64, 16
NC, NS = 2, 16
NW = NC * NS
CH = 128
K = 8
N_PAD = 10240
ROWS_BLK = 1000

_mesh = functools.partial(
    plsc.VectorSubcoreMesh, core_axis_name="c", subcore_axis_name="s")

_sc_params = pltpu.CompilerParams(use_tc_tiling_on_sc=False)


def _zero_vmem(buf, nwords):
    z16 = jnp.zeros((16,), jnp.float32)
    for i in range(nwords // 16):
        buf[pl.ds(i * 16, 16)] = z16


def _worker_prolog(zbuf, acc):
    s = lax.axis_index("s")
    zchunk = N_PAD // NS
    for i in range(zchunk // CH):
        pltpu.sync_copy(zbuf, acc.at[pl.ds(s * zchunk + i * CH, CH)])
    plsc.subcore_barrier()


def _make_deg(rows_per_worker):
    @functools.partial(
        pl.kernel,
        out_type=jax.ShapeDtypeStruct((NC * N_PAD,), jnp.float32),
        mesh=_mesh(),
        compiler_params=_sc_params,
        scratch_types=[
            pltpu.VMEM((K, CH), jnp.int32),
            pltpu.VMEM((CH,), jnp.float32),
            pltpu.VMEM((CH,), jnp.float32),
            pltpu.VMEM_SHARED((N_PAD,), jnp.float32),
        ],
    )
    def deg_kernel(col2d, degp, cidx, ones_v, sbuf, acc):
        c = lax.axis_index("c")
        s = lax.axis_index("s")
        wid = s * NC + c
        zchunk = N_PAD // NS
        for i in range(CH // 16):
            ones_v[pl.ds(i * 16, 16)] = jnp.ones((16,), jnp.float32)
        _zero_vmem(sbuf, CH)
        _worker_prolog(sbuf, acc)
        base = wid * rows_per_worker

        def step(t, carry):
            pltpu.sync_copy(col2d.at[pl.ds(base + t * K, K)], cidx)
            for j in range(K):
                pltpu.sync_copy(ones_v, acc.at[cidx.at[j]], add=True)
            return carry

        lax.fori_loop(0, rows_per_worker // K, step, 0)
        plsc.subcore_barrier()
        for i in range(zchunk // CH):
            pltpu.sync_copy(acc.at[pl.ds(s * zchunk + i * CH, CH)], sbuf)
            pltpu.sync_copy(
                sbuf, degp.at[pl.ds(c * N_PAD + s * zchunk + i * CH, CH)])

    return deg_kernel


def _make_msg(width, rows_per_worker):
    @functools.partial(
        pl.kernel,
        out_type=jax.ShapeDtypeStruct((NC, N_PAD, width), jnp.float32),
        mesh=_mesh(),
        compiler_params=_sc_params,
        scratch_types=[
            pltpu.VMEM((K, CH), jnp.int32),
            pltpu.VMEM((K, CH), jnp.int32),
            pltpu.VMEM((2, CH, width), jnp.float32),
            pltpu.VMEM_SHARED((N_PAD, width), jnp.float32),
            pltpu.SemaphoreType.DMA,
            pltpu.SemaphoreType.DMA,
        ],
    )
    def msg_kernel(table, row2d, col2d, raw,
                   ridx, cidx, gbuf, acc, sem0, sem1):
        c = lax.axis_index("c")
        s = lax.axis_index("s")
        wid = s * NC + c
        zchunk = N_PAD // NS

        def zrow(r, carry):
            z16 = jnp.zeros((16,), jnp.float32)
            for i in range(width // 16):
                gbuf[0, r, pl.ds(i * 16, 16)] = z16
            return carry

        lax.fori_loop(0, CH, zrow, 0)
        _worker_prolog(gbuf.at[0], acc)
        base = wid * rows_per_worker
        sems = [sem0, sem1]

        def step(t, carry):
            pltpu.sync_copy(row2d.at[pl.ds(base + t * K, K)], ridx)
            pltpu.sync_copy(col2d.at[pl.ds(base + t * K, K)], cidx)
            handles = [None, None]
            handles[0] = pltpu.async_copy(
                table.at[ridx.at[0]], gbuf.at[0], sems[0])
            for j in range(K):
                if j + 1 < K:
                    handles[(j + 1) % 2] = pltpu.async_copy(
                        table.at[ridx.at[j + 1]], gbuf.at[(j + 1) % 2],
                        sems[(j + 1) % 2])
                handles[j % 2].wait()
                pltpu.sync_copy(gbuf.at[j % 2], acc.at[cidx.at[j]], add=True)
            return carry

        lax.fori_loop(0, rows_per_worker // K, step, 0)
        plsc.subcore_barrier()
        for i in range(zchunk // CH):
            pltpu.sync_copy(acc.at[pl.ds(s * zchunk + i * CH, CH)],
                            gbuf.at[0])
            pltpu.sync_copy(gbuf.at[0],
                            raw.at[c, pl.ds(s * zchunk + i * CH, CH)])

    return msg_kernel


def _tc1_body(x_ref, w1_ref, degp_ref, hs1_ref, dis_ref):
    deg = degp_ref[0] + degp_ref[1] + 1.0
    dis = lax.rsqrt(deg)
    h = jnp.dot(x_ref[...], w1_ref[...], preferred_element_type=jnp.float32)
    hs1_ref[...] = dis * h
    dis_ref[...] = dis


def _tc2_body(raw1_ref, hs1_ref, dis_ref, b1_ref, w2_ref, hs2_ref):
    dis = dis_ref[...]
    h = jnp.maximum(
        dis * (raw1_ref[0] + raw1_ref[1] + hs1_ref[...]) + b1_ref[...], 0.0)
    hs2_ref[...] = dis * jnp.dot(
        h, w2_ref[...], preferred_element_type=jnp.float32)


def _tc3_body(raw2_ref, hs2_ref, dis_ref, b2_ref, out_ref):
    z = (dis_ref[...] * (raw2_ref[0] + raw2_ref[1] + hs2_ref[...])
         + b2_ref[...])
    m = jnp.max(z, axis=1, keepdims=True)
    ssum = jnp.sum(jnp.exp(z - m), axis=1, keepdims=True)
    out_ref[...] = z - m - jnp.log(ssum)


def _tc1(x, W1, degp_col):
    grid = N // ROWS_BLK
    return pl.pallas_call(
        _tc1_body,
        grid=(grid,),
        in_specs=[
            pl.BlockSpec((ROWS_BLK, D_IN), lambda i: (i, 0)),
            pl.BlockSpec((D_IN, D_H), lambda i: (0, 0)),
            pl.BlockSpec((NC, ROWS_BLK, 1), lambda i: (0, i, 0)),
        ],
        out_specs=[
            pl.BlockSpec((ROWS_BLK, D_H), lambda i: (i, 0)),
            pl.BlockSpec((ROWS_BLK, 1), lambda i: (i, 0)),
        ],
        out_shape=[
            jax.ShapeDtypeStruct((N, D_H), jnp.float32),
            jax.ShapeDtypeStruct((N, 1), jnp.float32),
        ],
    )(x, W1, degp_col)


def _tc2(raw1, hs1, dis, b1, W2):
    grid = N // ROWS_BLK
    return pl.pallas_call(
        _tc2_body,
        grid=(grid,),
        in_specs=[
            pl.BlockSpec((NC, ROWS_BLK, D_H), lambda i: (0, i, 0)),
            pl.BlockSpec((ROWS_BLK, D_H), lambda i: (i, 0)),
            pl.BlockSpec((ROWS_BLK, 1), lambda i: (i, 0)),
            pl.BlockSpec((1, D_H), lambda i: (0, 0)),
            pl.BlockSpec((D_H, D_OUT), lambda i: (0, 0)),
        ],
        out_specs=pl.BlockSpec((ROWS_BLK, D_OUT), lambda i: (i, 0)),
        out_shape=jax.ShapeDtypeStruct((N, D_OUT), jnp.float32),
    )(raw1, hs1, dis, b1, W2)


def _tc3(raw2, hs2, dis, b2):
    grid = N // ROWS_BLK
    return pl.pallas_call(
        _tc3_body,
        grid=(grid,),
        in_specs=[
            pl.BlockSpec((NC, ROWS_BLK, D_OUT), lambda i: (0, i, 0)),
            pl.BlockSpec((ROWS_BLK, D_OUT), lambda i: (i, 0)),
            pl.BlockSpec((ROWS_BLK, 1), lambda i: (i, 0)),
            pl.BlockSpec((1, D_OUT), lambda i: (0, 0)),
        ],
        out_specs=pl.BlockSpec((ROWS_BLK, D_OUT), lambda i: (i, 0)),
        out_shape=jax.ShapeDtypeStruct((N, D_OUT), jnp.float32),
    )(raw2, hs2, dis, b2)


def kernel(x, edge_index, W1, b1, W2, b2):
    row = edge_index[0]
    col = edge_index[1]
    e = row.shape[0]
    quantum = NW * K * CH
    e_pad = ((e + quantum - 1) // quantum) * quantum
    pad = e_pad - e
    rowp = jnp.concatenate([row, jnp.zeros((pad,), row.dtype)])
    colp = jnp.concatenate([col, jnp.full((pad,), N, col.dtype)])
    row2d = rowp.reshape(-1, CH)
    col2d = colp.reshape(-1, CH)
    rows_per_worker = row2d.shape[0] // NW

    degp = _make_deg(rows_per_worker)(col2d)
    degp_col = degp.reshape(NC, N_PAD, 1)[:, :N]

    hs1, dis = _tc1(x, W1, degp_col)
    raw1 = _make_msg(D_H, rows_per_worker)(hs1, row2d, col2d)[:, :N]
    hs2 = _tc2(raw1, hs1, dis, b1.reshape(1, D_H), W2)
    raw2 = _make_msg(D_OUT, rows_per_worker)(hs2, row2d, col2d)[:, :N]
    return _tc3(raw2, hs2, dis, b2.reshape(1, D_OUT))
